# Initial kernel scaffold; baseline (speedup 1.0000x reference)
import functools

import jax
import jax.numpy as jnp
from jax import lax
from jax.experimental import pallas as pl
from jax.experimental.pallas import tpu as pltpu
from jax.experimental.pallas import tpu_sc as plsc

_NUM_CORES = 2
_NUM_SUBCORES = 16
_NUM_WORKERS = _NUM_CORES * _NUM_SUBCORES


@functools.lru_cache(maxsize=None)
def _build_gather(num_rows: int, dim: int, batch: int):
    assert batch % _NUM_WORKERS == 0
    b_per_w = batch // _NUM_WORKERS
    chunk = min(1024, b_per_w)
    assert b_per_w % chunk == 0
    n_chunks = b_per_w // chunk

    mesh = plsc.VectorSubcoreMesh(core_axis_name="c", subcore_axis_name="s")

    @functools.partial(
        pl.kernel,
        mesh=mesh,
        out_type=jax.ShapeDtypeStruct((batch, dim), jnp.float32),
        scratch_types=[
            pltpu.VMEM((b_per_w,), jnp.int32),
            pltpu.VMEM((chunk, dim), jnp.float32),
            pltpu.SemaphoreType.DMA,
        ],
    )
    def gather_kernel(table_hbm, idx_hbm, out_hbm, idx_v, rows_v, sem):
        wid = lax.axis_index("s") * _NUM_CORES + lax.axis_index("c")
        base = wid * b_per_w
        pltpu.sync_copy(idx_hbm.at[pl.ds(base, b_per_w)], idx_v)

        def body(j, carry):
            off = j * chunk
            pltpu.async_copy(
                table_hbm.at[idx_v.at[pl.ds(off, chunk)]], rows_v, sem
            ).wait()
            pltpu.sync_copy(rows_v, out_hbm.at[pl.ds(base + off, chunk)])
            return carry

        lax.fori_loop(0, n_chunks, body, 0)

    return gather_kernel


def kernel(x, table):
    idx = x.reshape(-1).astype(jnp.int32)
    out = _build_gather(table.shape[0], table.shape[1], idx.shape[0])(table, idx)
    return out.reshape(x.shape + (table.shape[1],))


# speedup vs baseline: 1.1039x; 1.1039x over previous
import functools

import jax
import jax.numpy as jnp
from jax import lax
from jax.experimental import pallas as pl
from jax.experimental.pallas import tpu as pltpu
from jax.experimental.pallas import tpu_sc as plsc

_NUM_CORES = 2
_NUM_SUBCORES = 16
_NUM_WORKERS = _NUM_CORES * _NUM_SUBCORES


@functools.lru_cache(maxsize=None)
def _build_gather(num_rows: int, dim: int, batch: int):
    assert batch % _NUM_WORKERS == 0
    b_per_w = batch // _NUM_WORKERS
    chunk = min(1024, b_per_w)
    assert b_per_w % chunk == 0
    n_chunks = b_per_w // chunk

    mesh = plsc.VectorSubcoreMesh(core_axis_name="c", subcore_axis_name="s")

    @functools.partial(
        pl.kernel,
        mesh=mesh,
        out_type=jax.ShapeDtypeStruct((batch, dim), jnp.float32),
        scratch_types=[
            pltpu.VMEM((b_per_w,), jnp.int32),
            pltpu.VMEM((chunk, dim), jnp.float32),
            pltpu.SemaphoreType.DMA,
        ],
        compiler_params=pltpu.CompilerParams(use_tc_tiling_on_sc=False),
    )
    def gather_kernel(table_hbm, idx_hbm, out_hbm, idx_v, rows_v, sem):
        wid = lax.axis_index("s") * _NUM_CORES + lax.axis_index("c")
        base = wid * b_per_w
        pltpu.sync_copy(idx_hbm.at[pl.ds(base, b_per_w)], idx_v)

        def body(j, carry):
            off = j * chunk
            pltpu.async_copy(
                table_hbm.at[idx_v.at[pl.ds(off, chunk)]], rows_v, sem
            ).wait()
            pltpu.sync_copy(rows_v, out_hbm.at[pl.ds(base + off, chunk)])
            return carry

        lax.fori_loop(0, n_chunks, body, 0)

    return gather_kernel


def kernel(x, table):
    idx = x.reshape(-1).astype(jnp.int32)
    out = _build_gather(table.shape[0], table.shape[1], idx.shape[0])(table, idx)
    return out.reshape(x.shape + (table.shape[1],))


# speedup vs baseline: 1.1131x; 1.0083x over previous
import functools

import jax
import jax.numpy as jnp
from jax import lax
from jax.experimental import pallas as pl
from jax.experimental.pallas import tpu as pltpu
from jax.experimental.pallas import tpu_sc as plsc

_NUM_CORES = 2
_NUM_SUBCORES = 16
_NUM_WORKERS = _NUM_CORES * _NUM_SUBCORES

_CHUNK = 640
_NBUF = 4


@functools.lru_cache(maxsize=None)
def _build_gather(num_rows: int, dim: int, batch: int):
    assert batch % _NUM_WORKERS == 0
    b_per_w = batch // _NUM_WORKERS
    chunk = min(_CHUNK, b_per_w)
    assert b_per_w % chunk == 0
    n_slots = b_per_w // chunk
    assert n_slots % _NBUF == 0 and n_slots >= 2 * _NBUF
    n_groups = n_slots // _NBUF

    mesh = plsc.VectorSubcoreMesh(core_axis_name="c", subcore_axis_name="s")

    @functools.partial(
        pl.kernel,
        mesh=mesh,
        out_type=jax.ShapeDtypeStruct((batch, dim), jnp.float32),
        scratch_types=(
            [pltpu.VMEM((b_per_w,), jnp.int32)]
            + [pltpu.VMEM((chunk, dim), jnp.float32) for _ in range(_NBUF)]
            + [pltpu.SemaphoreType.DMA for _ in range(2 * _NBUF)]
        ),
        compiler_params=pltpu.CompilerParams(use_tc_tiling_on_sc=False),
    )
    def gather_kernel(table_hbm, idx_hbm, out_hbm, idx_v, *bufs_and_sems):
        bufs = bufs_and_sems[:_NBUF]
        gsem = bufs_and_sems[_NBUF : 2 * _NBUF]
        ssem = bufs_and_sems[2 * _NBUF :]
        wid = lax.axis_index("s") * _NUM_CORES + lax.axis_index("c")
        base = wid * b_per_w
        pltpu.sync_copy(idx_hbm.at[pl.ds(base, b_per_w)], idx_v)

        def start_gather(j, b):
            pltpu.async_copy(
                table_hbm.at[idx_v.at[pl.ds(j * chunk, chunk)]], bufs[b], gsem[b]
            )

        def wait_gather(b):
            pltpu.make_async_copy(
                table_hbm.at[idx_v.at[pl.ds(0, chunk)]], bufs[b], gsem[b]
            ).wait()

        def start_store(j, b):
            pltpu.async_copy(
                bufs[b], out_hbm.at[pl.ds(base + j * chunk, chunk)], ssem[b]
            )

        def wait_store(b):
            pltpu.make_async_copy(
                bufs[b], out_hbm.at[pl.ds(base, chunk)], ssem[b]
            ).wait()

        start_gather(0, 0)
        start_gather(1, 1)

        wait_gather(0)
        start_store(0, 0)
        start_gather(2, 2)
        wait_gather(1)
        start_store(1, 1)
        start_gather(3, 3)
        wait_gather(2)
        start_store(2, 2)
        wait_store(0)
        start_gather(4, 0)
        wait_gather(3)
        start_store(3, 3)
        wait_store(1)
        start_gather(5, 1)

        def group(g, carry):
            j0 = g * _NBUF
            for bs in range(_NBUF):
                j = j0 + bs
                nb = (bs + 2) % _NBUF
                wait_gather(bs)
                start_store(j, bs)
                wait_store(nb)
                start_gather(j + 2, nb)
            return carry

        lax.fori_loop(1, n_groups - 1, group, 0)

        j0 = n_slots - _NBUF
        wait_gather(0)
        start_store(j0 + 0, 0)
        wait_store(2)
        start_gather(j0 + 2, 2)
        wait_gather(1)
        start_store(j0 + 1, 1)
        wait_store(3)
        start_gather(j0 + 3, 3)
        wait_gather(2)
        start_store(j0 + 2, 2)
        wait_gather(3)
        start_store(j0 + 3, 3)
        for b in range(_NBUF):
            wait_store(b)

    return gather_kernel


def kernel(x, table):
    idx = x.reshape(-1).astype(jnp.int32)
    out = _build_gather(table.shape[0], table.shape[1], idx.shape[0])(table, idx)
    return out.reshape(x.shape + (table.shape[1],))


# speedup vs baseline: 5.0544x; 4.5407x over previous
import functools

import jax
import jax.numpy as jnp
from jax import lax
from jax.experimental import pallas as pl
from jax.experimental.pallas import tpu as pltpu
from jax.experimental.pallas import tpu_sc as plsc

_NUM_CORES = 2
_NUM_SUBCORES = 16
_S_CHUNK = 5


@functools.lru_cache(maxsize=None)
def _build_gather(num_rows: int, dim: int, seq: int, batch: int):
    assert dim % _NUM_CORES == 0
    d_per_core = dim // _NUM_CORES
    assert batch % _NUM_SUBCORES == 0
    b_chunk = batch // _NUM_SUBCORES
    assert seq % _S_CHUNK == 0
    n_chunks = seq // _S_CHUNK
    chunk = _S_CHUNK * b_chunk
    total = seq * b_chunk

    mesh = plsc.VectorSubcoreMesh(core_axis_name="c", subcore_axis_name="s")

    @functools.partial(
        pl.kernel,
        mesh=mesh,
        out_type=jax.ShapeDtypeStruct((seq, dim, batch), jnp.float32),
        scratch_types=[
            pltpu.VMEM((total,), jnp.int32),
            pltpu.VMEM((chunk,), jnp.float32),
            pltpu.VMEM((chunk,), jnp.float32),
            pltpu.VMEM_SHARED((num_rows,), jnp.float32),
            pltpu.SemaphoreType.DMA,
            pltpu.SemaphoreType.DMA,
            pltpu.SemaphoreType.DMA,
            pltpu.SemaphoreType.DMA,
        ],
    )
    def gather_kernel(tab_t, x_t, out, idx_v, buf0, buf1, colbuf,
                      gs0, gs1, ss0, ss1):
        cid = lax.axis_index("c")
        sid = lax.axis_index("s")
        b0 = sid * b_chunk
        gsems = (gs0, gs1)
        ssems = (ss0, ss1)
        bufs = (buf0, buf1)

        for s in range(seq):
            pltpu.async_copy(
                x_t.at[s, pl.ds(b0, b_chunk)],
                idx_v.at[pl.ds(s * b_chunk, b_chunk)],
                gsems[s % 2],
            )
        for s in range(seq):
            pltpu.make_async_copy(
                x_t.at[s, pl.ds(b0, b_chunk)],
                idx_v.at[pl.ds(0, b_chunk)],
                gsems[s % 2],
            ).wait()

        def wait_store_one(h):
            pltpu.make_async_copy(
                bufs[h].at[pl.ds(0, b_chunk)],
                out.at[0, 0, pl.ds(b0, b_chunk)],
                ssems[h],
            ).wait()

        def start_gather(q, h):
            pltpu.async_copy(
                colbuf.at[idx_v.at[pl.ds(q * chunk, chunk)]], bufs[h], gsems[h]
            )

        def wait_gather(h):
            pltpu.make_async_copy(
                colbuf.at[idx_v.at[pl.ds(0, chunk)]], bufs[h], gsems[h]
            ).wait()

        def feat(k, carry):
            d = cid * d_per_core + k
            plsc.subcore_barrier()

            @pl.when(sid == 0)
            def _():
                pltpu.sync_copy(tab_t.at[d], colbuf)

            plsc.subcore_barrier()
            for q in range(n_chunks):
                h = q % 2
                if q < 2:
                    @pl.when(k > 0)
                    def _():
                        for _ in range(_S_CHUNK):
                            wait_store_one(h)
                else:
                    for _ in range(_S_CHUNK):
                        wait_store_one(h)
                start_gather(q, h)
                if q > 0:
                    hp = (q - 1) % 2
                    wait_gather(hp)
                    for s in range(_S_CHUNK):
                        pltpu.async_copy(
                            bufs[hp].at[pl.ds(s * b_chunk, b_chunk)],
                            out.at[(q - 1) * _S_CHUNK + s, d, pl.ds(b0, b_chunk)],
                            ssems[hp],
                        )
            hl = (n_chunks - 1) % 2
            wait_gather(hl)
            for s in range(_S_CHUNK):
                pltpu.async_copy(
                    bufs[hl].at[pl.ds(s * b_chunk, b_chunk)],
                    out.at[(n_chunks - 1) * _S_CHUNK + s, d, pl.ds(b0, b_chunk)],
                    ssems[hl],
                )
            return carry

        lax.fori_loop(0, d_per_core, feat, 0)
        for h in range(2):
            for _ in range(_S_CHUNK):
                wait_store_one(h)

    return gather_kernel


def kernel(x, table):
    batch, seq = x.shape
    num_rows, dim = table.shape
    x_t = x.T.astype(jnp.int32)
    tab_t = table.T
    out_t = _build_gather(num_rows, dim, seq, batch)(tab_t, x_t)
    return jnp.transpose(out_t, (2, 0, 1))


# speedup vs baseline: 5.2369x; 1.0361x over previous
import functools

import jax
import jax.numpy as jnp
from jax import lax
from jax.experimental import pallas as pl
from jax.experimental.pallas import tpu as pltpu
from jax.experimental.pallas import tpu_sc as plsc

_NUM_CORES = 2
_NUM_SUBCORES = 16
_S_CHUNK = 2
_N_BUF = 5
_LAG = 3


@functools.lru_cache(maxsize=None)
def _build_gather(num_rows: int, dim: int, seq: int, batch: int):
    assert dim % _NUM_CORES == 0
    d_per_core = dim // _NUM_CORES
    assert batch % _NUM_SUBCORES == 0
    b_chunk = batch // _NUM_SUBCORES
    assert seq % _S_CHUNK == 0
    n_chunks = seq // _S_CHUNK
    assert n_chunks % _N_BUF == 0 and n_chunks > _N_BUF
    chunk = _S_CHUNK * b_chunk
    total = seq * b_chunk

    mesh = plsc.VectorSubcoreMesh(core_axis_name="c", subcore_axis_name="s")

    @functools.partial(
        pl.kernel,
        mesh=mesh,
        out_type=jax.ShapeDtypeStruct((seq, dim, batch), jnp.float32),
        scratch_types=(
            [pltpu.VMEM((total,), jnp.int32)]
            + [pltpu.VMEM((chunk,), jnp.float32) for _ in range(_N_BUF)]
            + [pltpu.VMEM_SHARED((num_rows,), jnp.float32)]
            + [pltpu.SemaphoreType.DMA for _ in range(2 * _N_BUF)]
        ),
    )
    def gather_kernel(tab_t, x_t, out, idx_v, *rest):
        bufs = rest[:_N_BUF]
        colbuf = rest[_N_BUF]
        gsems = rest[_N_BUF + 1 : 2 * _N_BUF + 1]
        ssems = rest[2 * _N_BUF + 1 :]
        cid = lax.axis_index("c")
        sid = lax.axis_index("s")
        b0 = sid * b_chunk

        for s in range(seq):
            pltpu.async_copy(
                x_t.at[s, pl.ds(b0, b_chunk)],
                idx_v.at[pl.ds(s * b_chunk, b_chunk)],
                gsems[s % _N_BUF],
            )
        for s in range(seq):
            pltpu.make_async_copy(
                x_t.at[s, pl.ds(b0, b_chunk)],
                idx_v.at[pl.ds(0, b_chunk)],
                gsems[s % _N_BUF],
            ).wait()

        def wait_store_one(h):
            pltpu.make_async_copy(
                bufs[h].at[pl.ds(0, b_chunk)],
                out.at[0, 0, pl.ds(b0, b_chunk)],
                ssems[h],
            ).wait()

        def start_gather(q, h):
            pltpu.async_copy(
                colbuf.at[idx_v.at[pl.ds(q * chunk, chunk)]], bufs[h], gsems[h]
            )

        def wait_gather(h):
            pltpu.make_async_copy(
                colbuf.at[idx_v.at[pl.ds(0, chunk)]], bufs[h], gsems[h]
            ).wait()

        def service(q, d):
            h = q % _N_BUF
            wait_gather(h)
            for s in range(_S_CHUNK):
                pltpu.async_copy(
                    bufs[h].at[pl.ds(s * b_chunk, b_chunk)],
                    out.at[q * _S_CHUNK + s, d, pl.ds(b0, b_chunk)],
                    ssems[h],
                )

        def feat(k, carry):
            d = cid * d_per_core + k
            plsc.subcore_barrier()

            @pl.when(sid == 0)
            def _():
                pltpu.sync_copy(tab_t.at[d], colbuf)

            plsc.subcore_barrier()
            for q in range(n_chunks):
                h = q % _N_BUF
                if q < _N_BUF:
                    @pl.when(k > 0)
                    def _():
                        for _ in range(_S_CHUNK):
                            wait_store_one(h)
                else:
                    for _ in range(_S_CHUNK):
                        wait_store_one(h)
                start_gather(q, h)
                if q >= _LAG:
                    service(q - _LAG, d)
            for q in range(n_chunks - _LAG, n_chunks):
                service(q, d)
            return carry

        lax.fori_loop(0, d_per_core, feat, 0)
        for h in range(_N_BUF):
            for _ in range(_S_CHUNK):
                wait_store_one(h)

    return gather_kernel


def kernel(x, table):
    batch, seq = x.shape
    num_rows, dim = table.shape
    x_t = x.T.astype(jnp.int32)
    tab_t = table.T
    out_t = _build_gather(num_rows, dim, seq, batch)(tab_t, x_t)
    return jnp.transpose(out_t, (2, 0, 1))
